# Initial kernel scaffold; baseline (speedup 1.0000x reference)
import jax
import jax.numpy as jnp
from jax.experimental import pallas as pl


def kernel(inputs, table):
    raise NotImplementedError("write your pallas kernel here")


# speedup vs baseline: 3.3692x; 3.3692x over previous
import jax
import jax.numpy as jnp
from jax.experimental import pallas as pl

B, L, D = 16384, 50, 128
ROWS = B * L
IDX_COLS = 128
S = 16


def _tc_body(idx_ref, table_ref, out_ref):
    t0 = table_ref[0:1, :]
    t1 = table_ref[1:2, :]
    ft = jnp.transpose(idx_ref[...])
    for s in range(S):
        col = ft[:, s:s + 1]
        out_ref[s] = jnp.where(col != 0, t1, t0)


def kernel(inputs, table):
    idx = inputs.reshape(ROWS // IDX_COLS, IDX_COLS)
    grid = (ROWS // IDX_COLS) // S
    out = pl.pallas_call(
        _tc_body,
        grid=(grid,),
        in_specs=[
            pl.BlockSpec((S, IDX_COLS), lambda i: (i, 0)),
            pl.BlockSpec((2, D), lambda i: (0, 0)),
        ],
        out_specs=pl.BlockSpec((S, IDX_COLS, D), lambda i: (i, 0, 0)),
        out_shape=jax.ShapeDtypeStruct((ROWS // IDX_COLS, IDX_COLS, D),
                                       jnp.float32),
    )(idx, table)
    return out.reshape(B, L, D)


# speedup vs baseline: 3.8991x; 1.1573x over previous
import jax
import jax.numpy as jnp
from jax.experimental import pallas as pl

B, L, D = 16384, 50, 128
ROWS = B * L
IDX_COLS = 128
S = 64


def _tc_body(idx_ref, table_ref, out_ref):
    t0 = table_ref[0:1, :]
    t1 = table_ref[1:2, :]
    ft = jnp.transpose(idx_ref[...])
    for s in range(S):
        col = ft[:, s:s + 1]
        out_ref[s] = jnp.where(col != 0, t1, t0)


def kernel(inputs, table):
    idx = inputs.reshape(ROWS // IDX_COLS, IDX_COLS)
    grid = (ROWS // IDX_COLS) // S
    out = pl.pallas_call(
        _tc_body,
        grid=(grid,),
        in_specs=[
            pl.BlockSpec((S, IDX_COLS), lambda i: (i, 0)),
            pl.BlockSpec((2, D), lambda i: (0, 0)),
        ],
        out_specs=pl.BlockSpec((S, IDX_COLS, D), lambda i: (i, 0, 0)),
        out_shape=jax.ShapeDtypeStruct((ROWS // IDX_COLS, IDX_COLS, D),
                                       jnp.float32),
    )(idx, table)
    return out.reshape(B, L, D)


# speedup vs baseline: 6.8721x; 1.7625x over previous
import jax
import jax.numpy as jnp
from jax.experimental import pallas as pl

B, L, D = 16384, 50, 128
BB = 64


def _tc_body(idx_ref, table_ref, out_ref):
    t0 = table_ref[0:1, :]
    t1 = table_ref[1:2, :]
    ft = jnp.transpose(idx_ref[...])
    for b in range(BB):
        col = ft[:, b:b + 1]
        out_ref[b] = jnp.where(col != 0, t1, t0)


def kernel(inputs, table):
    return pl.pallas_call(
        _tc_body,
        grid=(B // BB,),
        in_specs=[
            pl.BlockSpec((BB, L), lambda i: (i, 0)),
            pl.BlockSpec((2, D), lambda i: (0, 0)),
        ],
        out_specs=pl.BlockSpec((BB, L, D), lambda i: (i, 0, 0)),
        out_shape=jax.ShapeDtypeStruct((B, L, D), jnp.float32),
    )(inputs, table)


# speedup vs baseline: 7.8153x; 1.1373x over previous
import jax
import jax.numpy as jnp
from jax.experimental import pallas as pl

B, L, D = 16384, 50, 128
BB = 128


def _tc_body(idx_ref, table_ref, out_ref):
    t0 = table_ref[0:1, :]
    t1 = table_ref[1:2, :]
    ft = jnp.transpose(idx_ref[...])
    for b in range(BB):
        col = ft[:, b:b + 1]
        out_ref[b] = jnp.where(col != 0, t1, t0)


def kernel(inputs, table):
    return pl.pallas_call(
        _tc_body,
        grid=(B // BB,),
        in_specs=[
            pl.BlockSpec((BB, L), lambda i: (i, 0)),
            pl.BlockSpec((2, D), lambda i: (0, 0)),
        ],
        out_specs=pl.BlockSpec((BB, L, D), lambda i: (i, 0, 0)),
        out_shape=jax.ShapeDtypeStruct((B, L, D), jnp.float32),
    )(inputs, table)


# speedup vs baseline: 8.3096x; 1.0632x over previous
import jax
import jax.numpy as jnp
from jax.experimental import pallas as pl

B, L, D = 16384, 50, 128
BB = 256


def _tc_body(idx_ref, table_ref, out_ref):
    t0 = table_ref[0:1, :]
    t1 = table_ref[1:2, :]
    ft = jnp.transpose(idx_ref[...])
    for b in range(BB):
        col = ft[:, b:b + 1]
        out_ref[b] = jnp.where(col != 0, t1, t0)


def kernel(inputs, table):
    return pl.pallas_call(
        _tc_body,
        grid=(B // BB,),
        in_specs=[
            pl.BlockSpec((BB, L), lambda i: (i, 0)),
            pl.BlockSpec((2, D), lambda i: (0, 0)),
        ],
        out_specs=pl.BlockSpec((BB, L, D), lambda i: (i, 0, 0)),
        out_shape=jax.ShapeDtypeStruct((B, L, D), jnp.float32),
    )(inputs, table)


# speedup vs baseline: 8.4305x; 1.0146x over previous
import jax
import jax.numpy as jnp
from jax.experimental import pallas as pl

B, L, D = 16384, 50, 128
BB = 512


def _tc_body(idx_ref, table_ref, out_ref):
    t0 = table_ref[0:1, :]
    t1 = table_ref[1:2, :]
    ft = jnp.transpose(idx_ref[...])
    for b in range(BB):
        col = ft[:, b:b + 1]
        out_ref[b] = jnp.where(col != 0, t1, t0)


def kernel(inputs, table):
    return pl.pallas_call(
        _tc_body,
        grid=(B // BB,),
        in_specs=[
            pl.BlockSpec((BB, L), lambda i: (i, 0)),
            pl.BlockSpec((2, D), lambda i: (0, 0)),
        ],
        out_specs=pl.BlockSpec((BB, L, D), lambda i: (i, 0, 0)),
        out_shape=jax.ShapeDtypeStruct((B, L, D), jnp.float32),
    )(inputs, table)
